# baseline (speedup 1.0000x reference)
import functools

import jax
import jax.numpy as jnp
from jax import lax
from jax.experimental import pallas as pl
from jax.experimental.pallas import tpu as pltpu
from jax.experimental.pallas import tpu_sc as plsc

B = 16384
L = 50
D = 64
BT = B * L
NC = 2
NS = 16
NW = NC * NS
BPW = BT // NW
C = 200
NCH = BPW // C

_mesh = plsc.VectorSubcoreMesh(core_axis_name="c", subcore_axis_name="s")


@functools.partial(
    pl.kernel,
    mesh=_mesh,
    out_type=jax.ShapeDtypeStruct((BT, 2 * D), jnp.float32),
    scratch_types=[
        pltpu.VMEM((C,), jnp.int32),
        pltpu.VMEM((C, 16), jnp.int32),
        pltpu.VMEM((C, 2 * D), jnp.float32),
        pltpu.SemaphoreType.DMA,
    ],
)
def _gather_kernel(idx2_hbm, parx_hbm, table_hbm, out_hbm, idx2_v, parx_v,
                   rows_v, sem):
    wid = lax.axis_index("s") * NC + lax.axis_index("c")
    base = wid * BPW

    def body(g, carry):
        off = base + g * C
        pltpu.sync_copy(idx2_hbm.at[pl.ds(off, C)], idx2_v)
        pltpu.sync_copy(parx_hbm.at[pl.ds(off, C)], parx_v)
        pltpu.async_copy(table_hbm.at[idx2_v], rows_v, sem).wait()

        def sel_row(r, carry2):
            m = parx_v[r, pl.ds(0, 16)] != 0
            for k in range(D // 16):
                lo = rows_v[r, pl.ds(k * 16, 16)]
                hi = rows_v[r, pl.ds(D + k * 16, 16)]
                rows_v[r, pl.ds(k * 16, 16)] = jnp.where(m, hi, lo)
            return carry2

        lax.fori_loop(0, C, sel_row, 0)
        pltpu.sync_copy(rows_v, out_hbm.at[pl.ds(off, C)])
        return carry

    lax.fori_loop(0, NCH, body, 0)


def kernel(inputs, word_table):
    idx = inputs.reshape(BT).astype(jnp.int32)
    idx2 = jnp.right_shift(idx, 1)
    parx = jnp.broadcast_to((idx & 1)[:, None], (BT, 16))
    table2 = word_table.reshape(500000, 2 * D)
    out2 = _gather_kernel(idx2, parx, table2)
    return out2[:, :D].reshape(B, L, D)


# speedup vs baseline: 1.7972x; 1.7972x over previous
import functools

import jax
import jax.numpy as jnp
from jax import lax
from jax.experimental import pallas as pl
from jax.experimental.pallas import tpu as pltpu
from jax.experimental.pallas import tpu_sc as plsc

B = 16384
L = 50
D = 64
NC = 2
NS = 16
NW = NC * NS
BPW = B // NW
NBC = 16
NCH = BPW // NBC

_mesh = plsc.VectorSubcoreMesh(core_axis_name="c", subcore_axis_name="s")


@functools.partial(
    pl.kernel,
    mesh=_mesh,
    out_type=jax.ShapeDtypeStruct((B, L, D), jnp.float32),
    scratch_types=[
        pltpu.VMEM((NBC, L), jnp.int32),
        pltpu.VMEM((NBC, L, D), jnp.float32),
        pltpu.SemaphoreType.DMA,
    ],
    compiler_params=pltpu.CompilerParams(use_tc_tiling_on_sc=False),
)
def _gather_kernel(idx_hbm, table_hbm, out_hbm, idx_v, rows_v, sem):
    wid = lax.axis_index("s") * NC + lax.axis_index("c")
    b_base = wid * BPW

    def body(g, carry):
        bb = b_base + g * NBC
        pltpu.sync_copy(idx_hbm.at[pl.ds(bb, NBC)], idx_v)
        descs = [
            pltpu.async_copy(table_hbm.at[idx_v.at[b]], rows_v.at[b], sem)
            for b in range(NBC)
        ]
        for d in descs:
            d.wait()
        pltpu.sync_copy(rows_v, out_hbm.at[pl.ds(bb, NBC)])
        return carry

    lax.fori_loop(0, NCH, body, 0)


def kernel(inputs, word_table):
    return _gather_kernel(inputs.astype(jnp.int32), word_table)
